# baseline (speedup 1.0000x reference)
import functools

import jax
import jax.numpy as jnp
from jax import lax
from jax.experimental import pallas as pl
from jax.experimental.pallas import tpu as pltpu
from jax.experimental.pallas import tpu_sc as plsc

DM = 1024
NT = 8
NTOK = 2048
BLK = 128
LANES = 128
NBLK = 24
PAD = NBLK * BLK

RBLK = 512
NC = 2
NS = 16
NW = NC * NS
TPW = NTOK // NW


def _route_body(sig_ref, x_ref, ti_ref, dest_ref, gid_ref, act_ref, blkmap_ref,
                e_scr, rank_scr, carry_scr):
    j = pl.program_id(0)

    @pl.when(j == 0)
    def _():
        carry_scr[...] = jnp.zeros_like(carry_scr)

    @pl.when(j < NTOK // RBLK)
    def _pass1():
        sr = sig_ref[...]
        sigs = jnp.where(sr > 0.3, 1.0, jnp.where(sr < -0.3, -1.0, 0.0))
        xb = x_ref[...]
        scores = lax.dot_general(
            xb, sigs, (((1,), (1,)), ((), ())),
            preferred_element_type=jnp.float32)
        lane8 = lax.broadcasted_iota(jnp.int32, scores.shape, 1)
        m = jnp.max(scores, axis=1, keepdims=True)
        e = jnp.min(jnp.where(scores == m, lane8, NT), axis=1, keepdims=True)
        ti_ref[0] = e.reshape(RBLK // LANES, LANES)
        lane = lax.broadcasted_iota(jnp.int32, (RBLK, LANES), 1)
        onehot = (lane == e).astype(jnp.bfloat16)
        r_io = lax.broadcasted_iota(jnp.int32, (RBLK, RBLK), 0)
        c_io = lax.broadcasted_iota(jnp.int32, (RBLK, RBLK), 1)
        tri = (c_io < r_io).astype(jnp.bfloat16)
        ranks = lax.dot_general(
            tri, onehot, (((1,), (0,)), ((), ())),
            preferred_element_type=jnp.float32)
        ranks = ranks + carry_scr[...]
        rank = jnp.sum(ranks * onehot.astype(jnp.float32), axis=1, keepdims=True)
        e_scr[j] = e.reshape(RBLK // LANES, LANES)
        rank_scr[j] = rank.astype(jnp.int32).reshape(RBLK // LANES, LANES)
        carry_scr[...] = carry_scr[...] + jnp.sum(
            onehot.astype(jnp.float32), axis=0, keepdims=True)

    @pl.when(j == NTOK // RBLK)
    def _pass2():
        c = carry_scr[...]
        cpad = jnp.ceil(c * (1.0 / BLK)) * BLK
        f_io = lax.broadcasted_iota(jnp.int32, (LANES, LANES), 0)
        e_io = lax.broadcasted_iota(jnp.int32, (LANES, LANES), 1)
        t2 = (f_io < e_io).astype(jnp.float32)
        off = lax.dot_general(
            cpad, t2, (((1,), (0,)), ((), ())),
            preferred_element_type=jnp.float32)
        e = e_scr[...].reshape(NTOK // LANES, LANES)
        rank = rank_scr[...].reshape(NTOK // LANES, LANES)
        off_sel = jnp.zeros_like(rank)
        for t in range(NT):
            off_t = off[:, t:t + 1].astype(jnp.int32)
            off_sel = jnp.where(e == t, off_t, off_sel)
        dest = off_sel + rank
        dest_ref[...] = dest
        db = dest // BLK
        acc_cols = []
        for t in range(NBLK):
            acc_cols.append(jnp.max(jnp.where(db == t, e + 1, 0)))
        acc = jnp.stack(acc_cols).reshape(1, NBLK)
        acc = jnp.pad(acc, ((0, 0), (0, LANES - NBLK)))
        maxe = jnp.max(acc, axis=1, keepdims=True) - 1
        gid_ref[...] = jnp.where(acc > 0, acc - 1, maxe)
        actv = (acc > 0).astype(jnp.int32)
        act_ref[...] = actv
        nact = jnp.sum(actv, axis=1, keepdims=True)
        lane1 = lax.broadcasted_iota(jnp.int32, (1, LANES), 1)
        blkmap_ref[...] = jnp.minimum(lane1, nact - 1)


def _gmm_body(gid_ref, act_ref, blkmap_ref, xs_ref, w_ref, b_ref, o_ref):
    @pl.when(act_ref[0, pl.program_id(0)] > 0)
    def _():
        xb = xs_ref[...]
        y = lax.dot_general(
            xb, w_ref[0], (((1,), (1,)), ((), ())),
            preferred_element_type=jnp.float32)
        o_ref[...] = jnp.maximum(y + b_ref[0], 0.0) + xb


@functools.lru_cache(maxsize=1)
def _sc_kernels():
    mesh = plsc.VectorSubcoreMesh(core_axis_name="c", subcore_axis_name="s")

    @functools.partial(
        pl.kernel, mesh=mesh,
        out_type=jax.ShapeDtypeStruct((PAD, DM), jnp.float32),
        scratch_types=[pltpu.VMEM((TPW,), jnp.int32),
                       pltpu.VMEM((TPW, DM), jnp.float32),
                       pltpu.SemaphoreType.DMA],
    )
    def dispatch(x_hbm, dest_hbm, xs_hbm, idx_v, rows_v, sem):
        wid = lax.axis_index("s") * NC + lax.axis_index("c")
        base = wid * TPW
        pltpu.sync_copy(dest_hbm.at[pl.ds(base, TPW)], idx_v)
        pltpu.sync_copy(x_hbm.at[pl.ds(base, TPW)], rows_v)
        pltpu.async_copy(rows_v, xs_hbm.at[idx_v], sem).wait()

    @functools.partial(
        pl.kernel, mesh=mesh,
        out_type=jax.ShapeDtypeStruct((NTOK, DM), jnp.float32),
        scratch_types=[pltpu.VMEM((TPW,), jnp.int32),
                       pltpu.VMEM((TPW, DM), jnp.float32),
                       pltpu.SemaphoreType.DMA],
    )
    def collect(outs_hbm, dest_hbm, out_hbm, idx_v, rows_v, sem):
        wid = lax.axis_index("s") * NC + lax.axis_index("c")
        base = wid * TPW
        pltpu.sync_copy(dest_hbm.at[pl.ds(base, TPW)], idx_v)
        pltpu.async_copy(outs_hbm.at[idx_v], rows_v, sem).wait()
        pltpu.sync_copy(rows_v, out_hbm.at[pl.ds(base, TPW)])

    return dispatch, collect


def _route(x, sig_raw):
    return pl.pallas_call(
        _route_body,
        grid=(NTOK // RBLK + 1,),
        in_specs=[pl.BlockSpec((NT, DM), lambda j: (0, 0)),
                  pl.BlockSpec((RBLK, DM), lambda j: (jnp.minimum(j, NTOK // RBLK - 1), 0))],
        out_specs=[pl.BlockSpec((1, RBLK // LANES, LANES),
                                lambda j: (jnp.minimum(j, NTOK // RBLK - 1), 0, 0)),
                   pl.BlockSpec((NTOK // LANES, LANES), lambda j: (0, 0)),
                   pl.BlockSpec((1, LANES), lambda j: (0, 0)),
                   pl.BlockSpec((1, LANES), lambda j: (0, 0)),
                   pl.BlockSpec((1, LANES), lambda j: (0, 0))],
        out_shape=[jax.ShapeDtypeStruct((NTOK // RBLK, RBLK // LANES, LANES), jnp.int32),
                   jax.ShapeDtypeStruct((NTOK // LANES, LANES), jnp.int32),
                   jax.ShapeDtypeStruct((1, LANES), jnp.int32),
                   jax.ShapeDtypeStruct((1, LANES), jnp.int32),
                   jax.ShapeDtypeStruct((1, LANES), jnp.int32)],
        scratch_shapes=[pltpu.VMEM((NTOK // RBLK, RBLK // LANES, LANES), jnp.int32),
                        pltpu.VMEM((NTOK // RBLK, RBLK // LANES, LANES), jnp.int32),
                        pltpu.VMEM((1, LANES), jnp.float32)],
    )(sig_raw, x)


def _gmm(gid, act, blkmap, xs, W, b3):
    grid_spec = pltpu.PrefetchScalarGridSpec(
        num_scalar_prefetch=3,
        grid=(NBLK,),
        in_specs=[pl.BlockSpec((BLK, DM), lambda j, g, a, bm: (bm[0, j], 0)),
                  pl.BlockSpec((1, DM, DM), lambda j, g, a, bm: (g[0, j], 0, 0)),
                  pl.BlockSpec((1, 1, DM), lambda j, g, a, bm: (g[0, j], 0, 0))],
        out_specs=pl.BlockSpec((BLK, DM), lambda j, g, a, bm: (bm[0, j], 0)),
    )
    return pl.pallas_call(
        _gmm_body, grid_spec=grid_spec,
        out_shape=jax.ShapeDtypeStruct((PAD, DM), jnp.float32),
    )(gid, act, blkmap, xs, W, b3)


def kernel(x, sig_raw, W, b):
    ti_col, dest_col, gid_row, act_row, blkmap_row = _route(x, sig_raw)
    dest = dest_col.reshape(NTOK)
    dispatch, collect = _sc_kernels()
    xs = dispatch(x, dest)
    outs = _gmm(gid_row, act_row, blkmap_row, xs, W, b.reshape(NT, 1, DM))
    out = collect(outs, dest)
    return (out, ti_col.reshape(NTOK))


# speedup vs baseline: 1.1819x; 1.1819x over previous
import functools

import jax
import jax.numpy as jnp
from jax import lax
from jax.experimental import pallas as pl
from jax.experimental.pallas import tpu as pltpu
from jax.experimental.pallas import tpu_sc as plsc

DM = 1024
NT = 8
NTOK = 2048
BLK = 256
LANES = 128
NBLK = 16
PAD = NBLK * BLK

RBLK = 512
NC = 2
NS = 16
NW = NC * NS
TPW = NTOK // NW


def _route_body(sig_ref, x_ref, ti_ref, dest_ref, gid_ref, act_ref, blkmap_ref,
                e_scr, rank_scr, carry_scr):
    j = pl.program_id(0)

    @pl.when(j == 0)
    def _():
        carry_scr[...] = jnp.zeros_like(carry_scr)

    @pl.when(j < NTOK // RBLK)
    def _pass1():
        sr = sig_ref[...]
        sigs = jnp.where(sr > 0.3, 1.0, jnp.where(sr < -0.3, -1.0, 0.0))
        xb = x_ref[...]
        scores = lax.dot_general(
            xb, sigs, (((1,), (1,)), ((), ())),
            preferred_element_type=jnp.float32)
        lane8 = lax.broadcasted_iota(jnp.int32, scores.shape, 1)
        m = jnp.max(scores, axis=1, keepdims=True)
        e = jnp.min(jnp.where(scores == m, lane8, NT), axis=1, keepdims=True)
        ti_ref[0] = e.reshape(RBLK // LANES, LANES)
        lane = lax.broadcasted_iota(jnp.int32, (RBLK, LANES), 1)
        onehot = (lane == e).astype(jnp.bfloat16)
        r_io = lax.broadcasted_iota(jnp.int32, (RBLK, RBLK), 0)
        c_io = lax.broadcasted_iota(jnp.int32, (RBLK, RBLK), 1)
        tri = (c_io < r_io).astype(jnp.bfloat16)
        ranks = lax.dot_general(
            tri, onehot, (((1,), (0,)), ((), ())),
            preferred_element_type=jnp.float32)
        ranks = ranks + carry_scr[...]
        rank = jnp.sum(ranks * onehot.astype(jnp.float32), axis=1, keepdims=True)
        e_scr[j] = e.reshape(RBLK // LANES, LANES)
        rank_scr[j] = rank.astype(jnp.int32).reshape(RBLK // LANES, LANES)
        carry_scr[...] = carry_scr[...] + jnp.sum(
            onehot.astype(jnp.float32), axis=0, keepdims=True)

    @pl.when(j == NTOK // RBLK)
    def _pass2():
        c = carry_scr[...]
        cpad = jnp.ceil(c * (1.0 / BLK)) * BLK
        f_io = lax.broadcasted_iota(jnp.int32, (LANES, LANES), 0)
        e_io = lax.broadcasted_iota(jnp.int32, (LANES, LANES), 1)
        t2 = (f_io < e_io).astype(jnp.float32)
        off = lax.dot_general(
            cpad, t2, (((1,), (0,)), ((), ())),
            preferred_element_type=jnp.float32)
        e = e_scr[...].reshape(NTOK // LANES, LANES)
        rank = rank_scr[...].reshape(NTOK // LANES, LANES)
        off_sel = jnp.zeros_like(rank)
        for t in range(NT):
            off_t = off[:, t:t + 1].astype(jnp.int32)
            off_sel = jnp.where(e == t, off_t, off_sel)
        dest = off_sel + rank
        dest_ref[...] = dest
        db = dest // BLK
        acc_cols = []
        for t in range(NBLK):
            acc_cols.append(jnp.max(jnp.where(db == t, e + 1, 0)))
        acc = jnp.stack(acc_cols).reshape(1, NBLK)
        acc = jnp.pad(acc, ((0, 0), (0, LANES - NBLK)))
        maxe = jnp.max(acc, axis=1, keepdims=True) - 1
        gid_ref[...] = jnp.where(acc > 0, acc - 1, maxe)
        actv = (acc > 0).astype(jnp.int32)
        act_ref[...] = actv
        nact = jnp.sum(actv, axis=1, keepdims=True)
        lane1 = lax.broadcasted_iota(jnp.int32, (1, LANES), 1)
        blkmap_ref[...] = jnp.minimum(lane1, nact - 1)


def _gmm_body(gid_ref, act_ref, blkmap_ref, xs_ref, w_ref, b_ref, o_ref):
    @pl.when(act_ref[0, pl.program_id(0)] > 0)
    def _():
        xb = xs_ref[...]
        y = lax.dot_general(
            xb, w_ref[0], (((1,), (1,)), ((), ())),
            preferred_element_type=jnp.float32)
        o_ref[...] = jnp.maximum(y + b_ref[0], 0.0) + xb


@functools.lru_cache(maxsize=1)
def _sc_kernels():
    mesh = plsc.VectorSubcoreMesh(core_axis_name="c", subcore_axis_name="s")

    @functools.partial(
        pl.kernel, mesh=mesh,
        out_type=jax.ShapeDtypeStruct((PAD, DM), jnp.float32),
        scratch_types=[pltpu.VMEM((TPW,), jnp.int32),
                       pltpu.VMEM((TPW, DM), jnp.float32),
                       pltpu.SemaphoreType.DMA],
    )
    def dispatch(x_hbm, dest_hbm, xs_hbm, idx_v, rows_v, sem):
        wid = lax.axis_index("s") * NC + lax.axis_index("c")
        base = wid * TPW
        pltpu.sync_copy(dest_hbm.at[pl.ds(base, TPW)], idx_v)
        pltpu.sync_copy(x_hbm.at[pl.ds(base, TPW)], rows_v)
        pltpu.async_copy(rows_v, xs_hbm.at[idx_v], sem).wait()

    @functools.partial(
        pl.kernel, mesh=mesh,
        out_type=jax.ShapeDtypeStruct((NTOK, DM), jnp.float32),
        scratch_types=[pltpu.VMEM((TPW,), jnp.int32),
                       pltpu.VMEM((TPW, DM), jnp.float32),
                       pltpu.SemaphoreType.DMA],
    )
    def collect(outs_hbm, dest_hbm, out_hbm, idx_v, rows_v, sem):
        wid = lax.axis_index("s") * NC + lax.axis_index("c")
        base = wid * TPW
        pltpu.sync_copy(dest_hbm.at[pl.ds(base, TPW)], idx_v)
        pltpu.async_copy(outs_hbm.at[idx_v], rows_v, sem).wait()
        pltpu.sync_copy(rows_v, out_hbm.at[pl.ds(base, TPW)])

    return dispatch, collect


def _route(x, sig_raw):
    return pl.pallas_call(
        _route_body,
        grid=(NTOK // RBLK + 1,),
        in_specs=[pl.BlockSpec((NT, DM), lambda j: (0, 0)),
                  pl.BlockSpec((RBLK, DM), lambda j: (jnp.minimum(j, NTOK // RBLK - 1), 0))],
        out_specs=[pl.BlockSpec((1, RBLK // LANES, LANES),
                                lambda j: (jnp.minimum(j, NTOK // RBLK - 1), 0, 0)),
                   pl.BlockSpec((NTOK // LANES, LANES), lambda j: (0, 0)),
                   pl.BlockSpec((1, LANES), lambda j: (0, 0)),
                   pl.BlockSpec((1, LANES), lambda j: (0, 0)),
                   pl.BlockSpec((1, LANES), lambda j: (0, 0))],
        out_shape=[jax.ShapeDtypeStruct((NTOK // RBLK, RBLK // LANES, LANES), jnp.int32),
                   jax.ShapeDtypeStruct((NTOK // LANES, LANES), jnp.int32),
                   jax.ShapeDtypeStruct((1, LANES), jnp.int32),
                   jax.ShapeDtypeStruct((1, LANES), jnp.int32),
                   jax.ShapeDtypeStruct((1, LANES), jnp.int32)],
        scratch_shapes=[pltpu.VMEM((NTOK // RBLK, RBLK // LANES, LANES), jnp.int32),
                        pltpu.VMEM((NTOK // RBLK, RBLK // LANES, LANES), jnp.int32),
                        pltpu.VMEM((1, LANES), jnp.float32)],
    )(sig_raw, x)


def _gmm(gid, act, blkmap, xs, W, b3):
    grid_spec = pltpu.PrefetchScalarGridSpec(
        num_scalar_prefetch=3,
        grid=(NBLK,),
        in_specs=[pl.BlockSpec((BLK, DM), lambda j, g, a, bm: (bm[0, j], 0)),
                  pl.BlockSpec((1, DM, DM), lambda j, g, a, bm: (g[0, j], 0, 0)),
                  pl.BlockSpec((1, 1, DM), lambda j, g, a, bm: (g[0, j], 0, 0))],
        out_specs=pl.BlockSpec((BLK, DM), lambda j, g, a, bm: (bm[0, j], 0)),
    )
    return pl.pallas_call(
        _gmm_body, grid_spec=grid_spec,
        out_shape=jax.ShapeDtypeStruct((PAD, DM), jnp.float32),
    )(gid, act, blkmap, xs, W, b3)


def kernel(x, sig_raw, W, b):
    ti_col, dest_col, gid_row, act_row, blkmap_row = _route(x, sig_raw)
    dest = dest_col.reshape(NTOK)
    dispatch, collect = _sc_kernels()
    xs = dispatch(x, dest)
    outs = _gmm(gid_row, act_row, blkmap_row, xs, W, b.reshape(NT, 1, DM))
    out = collect(outs, dest)
    return (out, ti_col.reshape(NTOK))
